# Initial kernel scaffold; baseline (speedup 1.0000x reference)
import jax
import jax.numpy as jnp
from jax.experimental import pallas as pl


def kernel(x, edge_index, edge_weight, edge_attr, colors, mlp_W1, mlp_b1, mlp_W2, mlp_b2, lin1_W, lin2_W, lin2_b, lin_W, lin_b):
    raise NotImplementedError("write your pallas kernel here")


# speedup vs baseline: 1.2493x; 1.2493x over previous
import functools
from math import pi as PI

import jax
import jax.numpy as jnp
from jax import lax
from jax.experimental import pallas as pl
from jax.experimental.pallas import tpu as pltpu
from jax.experimental.pallas import tpu_sc as plsc

CUTOFF = 10.0
LOG2 = 0.6931471805599453

N = 10000
E = 160000
H = 256
G = 50
GP = 64
FNUM = 256
NC = 4
HH = H // 2

BE = 800
BN = 1000

NUM_TILES = 16
EP = E // NUM_TILES
KCH = 80
NPAD = 10240
NP = NPAD // NUM_TILES
SB = 128


def _ssp(v):
    return jax.nn.softplus(v) - LOG2


def _filters_body(ea_ref, cf_ref, ew_ref, w1_ref, b1_ref, w2_ref, b2_ref,
                  o0_ref, o1_ref):
    ea = ea_ref[...]
    cf = cf_ref[...]
    c_env = 0.5 * (jnp.cos(ew_ref[...] * (PI / CUTOFF)) + 1.0)
    acc = jnp.zeros((BE, FNUM), jnp.float32)
    for c in range(NC):
        h1 = _ssp(jnp.dot(ea, w1_ref[c], preferred_element_type=jnp.float32)
                  + b1_ref[0, c])
        f = jnp.dot(h1, w2_ref[c], preferred_element_type=jnp.float32) + b2_ref[0, c]
        acc = jnp.where(cf == float(c), f, acc)
    acc = acc * c_env
    o0_ref[...] = acc[:, :HH]
    o1_ref[...] = acc[:, HH:]


def _filters(ea_pad, colors_f, ew_col, w1, b1, w2, b2):
    grid = (E // BE,)
    return pl.pallas_call(
        _filters_body,
        grid=grid,
        in_specs=[
            pl.BlockSpec((BE, GP), lambda i: (i, 0)),
            pl.BlockSpec((BE, 1), lambda i: (i, 0)),
            pl.BlockSpec((BE, 1), lambda i: (i, 0)),
            pl.BlockSpec((NC, GP, FNUM), lambda i: (0, 0, 0)),
            pl.BlockSpec((1, NC, FNUM), lambda i: (0, 0, 0)),
            pl.BlockSpec((NC, FNUM, FNUM), lambda i: (0, 0, 0)),
            pl.BlockSpec((1, NC, FNUM), lambda i: (0, 0, 0)),
        ],
        out_specs=[
            pl.BlockSpec((BE, HH), lambda i: (i, 0)),
            pl.BlockSpec((BE, HH), lambda i: (i, 0)),
        ],
        out_shape=[
            jax.ShapeDtypeStruct((E, HH), jnp.float32),
            jax.ShapeDtypeStruct((E, HH), jnp.float32),
        ],
    )(ea_pad, colors_f, ew_col, w1, b1, w2, b2)


def _lin1_body(x_ref, w_ref, o0_ref, o1_ref):
    h = jnp.dot(x_ref[...], w_ref[...], preferred_element_type=jnp.float32)
    o0_ref[...] = h[:, :HH]
    o1_ref[...] = h[:, HH:]


def _lin1(x, lin1_W):
    return pl.pallas_call(
        _lin1_body,
        grid=(N // BN,),
        in_specs=[
            pl.BlockSpec((BN, H), lambda i: (i, 0)),
            pl.BlockSpec((H, FNUM), lambda i: (0, 0)),
        ],
        out_specs=[
            pl.BlockSpec((BN, HH), lambda i: (i, 0)),
            pl.BlockSpec((BN, HH), lambda i: (i, 0)),
        ],
        out_shape=[
            jax.ShapeDtypeStruct((N, HH), jnp.float32),
            jax.ShapeDtypeStruct((N, HH), jnp.float32),
        ],
    )(x, lin1_W)


def _zero_rows(buf, rows):
    z = jnp.zeros((16,), jnp.float32)

    def zrow(r, carry):
        for c8 in range(HH // 16):
            buf[r, pl.ds(c8 * 16, 16)] = z
        return carry

    lax.fori_loop(0, rows, zrow, 0)


def _msg_half(h_hbm, w_hbm, src_hbm, dst_hbm, agg_hbm,
              srcbuf, dstbuf, hrows, wrows, sbuf, acc, sem1, sem2, sid):
    base = sid * EP

    def chunk(k, carry):
        off = base + k * KCH
        pltpu.sync_copy(src_hbm.at[pl.ds(off, KCH)], srcbuf)
        pltpu.sync_copy(dst_hbm.at[pl.ds(off, KCH)], dstbuf)
        cp1 = pltpu.async_copy(h_hbm.at[srcbuf], hrows, sem1)
        cp2 = pltpu.async_copy(w_hbm.at[pl.ds(off, KCH)], wrows, sem2)
        cp1.wait()
        cp2.wait()

        def mrow(r, c2):
            for c8 in range(HH // 16):
                s = pl.ds(c8 * 16, 16)
                wrows[r, s] = wrows[r, s] * hrows[r, s]
            return c2

        lax.fori_loop(0, KCH, mrow, 0)
        pltpu.sync_copy(wrows, acc.at[dstbuf], add=True)
        return carry

    lax.fori_loop(0, EP // KCH, chunk, 0)
    plsc.subcore_barrier()
    for r in range(NP // SB):
        sl = pl.ds(sid * NP + r * SB, SB)
        cp = pltpu.async_copy(acc.at[sl], sbuf, sem1)
        cp.wait()
        pltpu.sync_copy(sbuf, agg_hbm.at[sl])


def _msg_kernel_body(h0, h1, w0, w1, src_hbm, dst_hbm, agg0, agg1,
                     srcbuf, dstbuf, hrows, wrows, sbuf, acc, sem1, sem2):
    cid = lax.axis_index("c")
    sid = lax.axis_index("s")
    _zero_rows(sbuf, SB)
    for r in range(NP // SB):
        pltpu.sync_copy(sbuf, acc.at[pl.ds(sid * NP + r * SB, SB)])
    plsc.subcore_barrier()

    @pl.when(cid == 0)
    def _():
        _msg_half(h0, w0, src_hbm, dst_hbm, agg0,
                  srcbuf, dstbuf, hrows, wrows, sbuf, acc, sem1, sem2, sid)

    @pl.when(cid == 1)
    def _():
        _msg_half(h1, w1, src_hbm, dst_hbm, agg1,
                  srcbuf, dstbuf, hrows, wrows, sbuf, acc, sem1, sem2, sid)


def _msg_agg(h0, h1, w0, w1, src, dst):
    mesh = plsc.VectorSubcoreMesh(core_axis_name="c", subcore_axis_name="s")
    f = pl.kernel(
        _msg_kernel_body,
        out_type=[
            jax.ShapeDtypeStruct((NPAD, HH), jnp.float32),
            jax.ShapeDtypeStruct((NPAD, HH), jnp.float32),
        ],
        mesh=mesh,
        scratch_types=[
            pltpu.VMEM((KCH,), jnp.int32),
            pltpu.VMEM((KCH,), jnp.int32),
            pltpu.VMEM((KCH, HH), jnp.float32),
            pltpu.VMEM((KCH, HH), jnp.float32),
            pltpu.VMEM((SB, HH), jnp.float32),
            pltpu.VMEM_SHARED((NPAD, HH), jnp.float32),
            pltpu.SemaphoreType.DMA,
            pltpu.SemaphoreType.DMA,
        ],
    )
    return f(h0, h1, w0, w1, src, dst)


def _tail_body(a0_ref, a1_ref, w2a_ref, w2b_ref, b2_ref, w_ref, b_ref, o_ref):
    t = (jnp.dot(a0_ref[...], w2a_ref[...], preferred_element_type=jnp.float32)
         + jnp.dot(a1_ref[...], w2b_ref[...], preferred_element_type=jnp.float32)
         + b2_ref[...])
    t = _ssp(t)
    o_ref[...] = jnp.dot(t, w_ref[...], preferred_element_type=jnp.float32) + b_ref[...]


def _tail(a0, a1, lin2_Wa, lin2_Wb, lin2_b, lin_W, lin_b):
    return pl.pallas_call(
        _tail_body,
        grid=(N // BN,),
        in_specs=[
            pl.BlockSpec((BN, HH), lambda i: (i, 0)),
            pl.BlockSpec((BN, HH), lambda i: (i, 0)),
            pl.BlockSpec((HH, H), lambda i: (0, 0)),
            pl.BlockSpec((HH, H), lambda i: (0, 0)),
            pl.BlockSpec((1, H), lambda i: (0, 0)),
            pl.BlockSpec((H, H), lambda i: (0, 0)),
            pl.BlockSpec((1, H), lambda i: (0, 0)),
        ],
        out_specs=pl.BlockSpec((BN, H), lambda i: (i, 0)),
        out_shape=jax.ShapeDtypeStruct((N, H), jnp.float32),
    )(a0, a1, lin2_Wa, lin2_Wb, lin2_b, lin_W, lin_b)


def kernel(x, edge_index, edge_weight, edge_attr, colors,
           mlp_W1, mlp_b1, mlp_W2, mlp_b2,
           lin1_W, lin2_W, lin2_b, lin_W, lin_b):
    assert x.shape == (N, H) and edge_attr.shape == (E, G)
    src = edge_index[0]
    dst = edge_index[1]
    ea_pad = jnp.pad(edge_attr, ((0, 0), (0, GP - G)))
    w1_pad = jnp.pad(mlp_W1, ((0, 0), (0, GP - G), (0, 0)))
    colors_f = colors.astype(jnp.float32)[:, None]
    ew_col = edge_weight[:, None]

    w0, wf1 = _filters(ea_pad, colors_f, ew_col,
                       w1_pad, mlp_b1[None], mlp_W2, mlp_b2[None])
    h0, h1 = _lin1(x, lin1_W)
    a0, a1 = _msg_agg(h0, h1, w0, wf1, src, dst)
    out = _tail(a0, a1, lin2_W[:HH], lin2_W[HH:], lin2_b[None],
                lin_W, lin_b[None])
    return out
